# baseline (speedup 1.0000x reference)
import functools

import jax
import jax.numpy as jnp
from jax import lax
from jax.experimental import pallas as pl
from jax.experimental.pallas import tpu as pltpu
from jax.experimental.pallas import tpu_sc as plsc

N = 10000
E = 320000
D_IN, D_HID, D_OUT = 128, 32, 16

NC, NS = 2, 16
CHUNK = 128
NBUF = 6
TOT_CHUNK = E // CHUNK
K0 = 66
K1 = 90
XTRA = 4
KMAX = max(K0 + 1, K1)
KD = 156
ROWS = 632
ACC_N = ROWS * NS
TAIL = N - (NS - 1) * ROWS
DEGW = 8
ZB = 64

_mesh = plsc.VectorSubcoreMesh(core_axis_name="c", subcore_axis_name="s")
_sc_params = pltpu.CompilerParams(use_tc_tiling_on_sc=False)


def _copy_out(acc, out_hbm, s):
  @pl.when(s < NS - 1)
  def _():
    pltpu.sync_copy(acc.at[pl.ds(s * ROWS, ROWS)],
                    out_hbm.at[pl.ds(s * ROWS, ROWS)])

  @pl.when(s == NS - 1)
  def _():
    pltpu.sync_copy(acc.at[pl.ds((NS - 1) * ROWS, TAIL)],
                    out_hbm.at[pl.ds((NS - 1) * ROWS, TAIL)])


def _zero_acc_local(zbuf, acc, s, d):
  z16 = jnp.zeros((16,), jnp.float32)

  def zrow(r, carry):
    for cc in range(d // 16):
      zbuf[r, pl.ds(cc * 16, 16)] = z16
    return carry

  lax.fori_loop(0, ZB, zrow, 0)
  full, rem = divmod(ROWS, ZB)
  for i in range(full):
    pltpu.sync_copy(zbuf, acc.at[pl.ds(s * ROWS + i * ZB, ZB)])
  if rem:
    pltpu.sync_copy(zbuf.at[pl.ds(0, rem)],
                    acc.at[pl.ds(s * ROWS + full * ZB, rem)])


def _make_deg_kernel():
  @functools.partial(
      pl.kernel,
      out_type=jax.ShapeDtypeStruct((N, DEGW), jnp.float32),
      mesh=_mesh,
      compiler_params=_sc_params,
      name="sc_deg",
      scratch_types=[
          pltpu.VMEM((KD + 1, CHUNK), jnp.int32),
          pltpu.VMEM((CHUNK, DEGW), jnp.float32),
          pltpu.VMEM_SHARED((ACC_N, DEGW), jnp.float32),
          pltpu.SemaphoreType.DMA,
      ],
  )
  def deg_kernel(ei_hbm, ones_hbm, zeros_hbm, out_hbm, dst_v, ones_v, acc,
                 ssem):
    c = lax.axis_index("c")
    s = lax.axis_index("s")

    @pl.when(c == 0)
    def _():
      base = s * KD + jnp.minimum(s, XTRA)
      pltpu.sync_copy(zeros_hbm, acc.at[pl.ds(s * ROWS, ROWS)])
      pltpu.sync_copy(ones_hbm, ones_v)
      pltpu.sync_copy(ei_hbm.at[1, pl.ds(base, KD)], dst_v.at[pl.ds(0, KD)])

      @pl.when(s < XTRA)
      def _():
        pltpu.sync_copy(ei_hbm.at[1, pl.ds(base + KD, 1)],
                        dst_v.at[pl.ds(KD, 1)])

      plsc.subcore_barrier()

      def body(g, carry):
        for b in range(NBUF):
          pltpu.async_copy(ones_v, acc.at[dst_v.at[g * NBUF + b]], ssem,
                           add=True)
        for b in range(NBUF):
          pltpu.make_async_copy(ones_v, acc.at[pl.ds(0, CHUNK)], ssem).wait()
        return carry

      lax.fori_loop(0, KD // NBUF, body, 0)

      @pl.when(s < XTRA)
      def _():
        pltpu.sync_copy(ones_v, acc.at[dst_v.at[KD]], add=True)

      plsc.subcore_barrier()
      _copy_out(acc, out_hbm, s)

  return deg_kernel


def _make_scatter_kernel(d: int):
  @functools.partial(
      pl.kernel,
      out_type=(jax.ShapeDtypeStruct((N, d), jnp.float32),
                jax.ShapeDtypeStruct((N, d), jnp.float32)),
      mesh=_mesh,
      compiler_params=_sc_params,
      name=f"sc_scat{d}",
      scratch_types=[
          pltpu.VMEM((KMAX, CHUNK), jnp.int32),
          pltpu.VMEM((KMAX, CHUNK), jnp.int32),
          pltpu.VMEM((NBUF, CHUNK, d), jnp.float32),
          pltpu.VMEM((ZB, d), jnp.float32),
          pltpu.VMEM_SHARED((ACC_N, d), jnp.float32),
      ] + [pltpu.SemaphoreType.DMA] * (2 * NBUF),
  )
  def scat_kernel(z_hbm, ei_hbm, out0_hbm, out1_hbm,
                  src_v, dst_v, rows_v, zbuf, acc, *sems):
    gsems, ssems = sems[:NBUF], sems[NBUF:]
    c = lax.axis_index("c")
    s = lax.axis_index("s")

    def gather_start(b, j):
      pltpu.async_copy(z_hbm.at[src_v.at[j]], rows_v.at[b], gsems[b])

    def gather_wait(b):
      pltpu.make_async_copy(z_hbm.at[pl.ds(0, CHUNK)], rows_v.at[b],
                            gsems[b]).wait()

    def scatter_start(b, j):
      pltpu.async_copy(rows_v.at[b], acc.at[dst_v.at[j]], ssems[b], add=True)

    def scatter_wait(b):
      pltpu.make_async_copy(rows_v.at[b], acc.at[pl.ds(0, CHUNK)],
                            ssems[b]).wait()

    def run(base, nchunk):
      pltpu.sync_copy(ei_hbm.at[0, pl.ds(base, nchunk)],
                      src_v.at[pl.ds(0, nchunk)])
      pltpu.sync_copy(ei_hbm.at[1, pl.ds(base, nchunk)],
                      dst_v.at[pl.ds(0, nchunk)])
      plsc.subcore_barrier()
      ng = nchunk // NBUF
      for b in range(NBUF):
        gather_start(b, b)

      def body(g, carry):
        for b in range(NBUF):
          gather_wait(b)
          scatter_start(b, g * NBUF + b)
        for b in range(NBUF):
          scatter_wait(b)
          gather_start(b, (g + 1) * NBUF + b)
        return carry

      lax.fori_loop(0, ng - 1, body, 0)
      for b in range(NBUF):
        gather_wait(b)
        scatter_start(b, (ng - 1) * NBUF + b)
      for b in range(NBUF):
        scatter_wait(b)

    @pl.when(c == 0)
    def _():
      base = s * K0 + jnp.minimum(s, XTRA)
      _zero_acc_local(zbuf, acc, s, d)

      @pl.when(s < XTRA)
      def _():
        pltpu.sync_copy(ei_hbm.at[0, pl.ds(base + K0, 1)],
                        src_v.at[pl.ds(K0, 1)])
        pltpu.sync_copy(ei_hbm.at[1, pl.ds(base + K0, 1)],
                        dst_v.at[pl.ds(K0, 1)])

      run(base, K0)

      @pl.when(s < XTRA)
      def _():
        pltpu.sync_copy(z_hbm.at[src_v.at[K0]], rows_v.at[0])
        pltpu.sync_copy(rows_v.at[0], acc.at[dst_v.at[K0]], add=True)

      plsc.subcore_barrier()
      _copy_out(acc, out0_hbm, s)

    @pl.when(c == 1)
    def _():
      _zero_acc_local(zbuf, acc, s, d)
      run(NS * K0 + XTRA + s * K1, K1)
      plsc.subcore_barrier()
      _copy_out(acc, out1_hbm, s)

  return scat_kernel


_deg_kernel = _make_deg_kernel()
_scat32 = _make_scatter_kernel(D_HID)
_scat16 = _make_scatter_kernel(D_OUT)


def _tc1a(x_ref, w1_ref, y1_ref):
  y1_ref[...] = jnp.dot(x_ref[...], w1_ref[...],
                        preferred_element_type=jnp.float32)


def _tc1b(y1_ref, deg_ref, z1_ref, dinv_ref):
  deg = deg_ref[:, :1] + 1.0
  dinv = lax.rsqrt(deg)
  z1_ref[...] = y1_ref[...] * dinv
  dinv_ref[...] = dinv


def _tc2(s1a_ref, s1b_ref, z1_ref, dinv_ref, b1_ref, w2_ref, z2_ref):
  dinv = dinv_ref[...]
  h = dinv * (s1a_ref[...] + s1b_ref[...] + z1_ref[...]) + b1_ref[...]
  h = jnp.maximum(h, 0.0)
  y2 = jnp.dot(h, w2_ref[...], preferred_element_type=jnp.float32)
  z2_ref[...] = y2 * dinv


def _tc3(s2a_ref, s2b_ref, z2_ref, dinv_ref, b2_ref, out_ref):
  out_ref[...] = (dinv_ref[...] * (s2a_ref[...] + s2b_ref[...] + z2_ref[...])
                  + b2_ref[...])


def kernel(x, edge_index, W1, b1, W2, b2):
  ei3 = edge_index.reshape(2, TOT_CHUNK, CHUNK)

  ones_c = jnp.ones((CHUNK, DEGW), jnp.float32)
  zeros1 = jnp.zeros((ROWS, DEGW), jnp.float32)

  deg = _deg_kernel(ei3, ones_c, zeros1)

  y1 = pl.pallas_call(
      _tc1a,
      out_shape=jax.ShapeDtypeStruct((N, D_HID), jnp.float32),
  )(x, W1)

  z1, dinv = pl.pallas_call(
      _tc1b,
      out_shape=(jax.ShapeDtypeStruct((N, D_HID), jnp.float32),
                 jax.ShapeDtypeStruct((N, 1), jnp.float32)),
  )(y1, deg)

  s1a, s1b = _scat32(z1, ei3)

  z2 = pl.pallas_call(
      _tc2,
      out_shape=jax.ShapeDtypeStruct((N, D_OUT), jnp.float32),
  )(s1a, s1b, z1, dinv, b1, W2)

  s2a, s2b = _scat16(z2, ei3)

  out = pl.pallas_call(
      _tc3,
      out_shape=jax.ShapeDtypeStruct((N, D_OUT), jnp.float32),
  )(s2a, s2b, z2, dinv, b2)

  return out


# speedup vs baseline: 1.0117x; 1.0117x over previous
import functools

import jax
import jax.numpy as jnp
from jax import lax
from jax.experimental import pallas as pl
from jax.experimental.pallas import tpu as pltpu
from jax.experimental.pallas import tpu_sc as plsc

N = 10000
E = 320000
D_IN, D_HID, D_OUT = 128, 32, 16

NC, NS = 2, 16
CHUNK = 128
NBUF = 6
TOT_CHUNK = E // CHUNK
K0 = 66
K1 = 90
XTRA = 4
KMAX = max(K0 + 1, K1)
KD = 156
ROWS = 632
ACC_N = ROWS * NS
TAIL = N - (NS - 1) * ROWS
DEGW = 8
ZB = 64

_mesh = plsc.VectorSubcoreMesh(core_axis_name="c", subcore_axis_name="s")
_sc_params = pltpu.CompilerParams(use_tc_tiling_on_sc=False)


def _copy_out(acc, out_hbm, s):
  @pl.when(s < NS - 1)
  def _():
    pltpu.sync_copy(acc.at[pl.ds(s * ROWS, ROWS)],
                    out_hbm.at[pl.ds(s * ROWS, ROWS)])

  @pl.when(s == NS - 1)
  def _():
    pltpu.sync_copy(acc.at[pl.ds((NS - 1) * ROWS, TAIL)],
                    out_hbm.at[pl.ds((NS - 1) * ROWS, TAIL)])


def _zero_acc_local(zbuf, acc, s, d):
  z16 = jnp.zeros((16,), jnp.float32)

  def zrow(r, carry):
    for cc in range(d // 16):
      zbuf[r, pl.ds(cc * 16, 16)] = z16
    return carry

  lax.fori_loop(0, ZB, zrow, 0)
  full, rem = divmod(ROWS, ZB)
  for i in range(full):
    pltpu.sync_copy(zbuf, acc.at[pl.ds(s * ROWS + i * ZB, ZB)])
  if rem:
    pltpu.sync_copy(zbuf.at[pl.ds(0, rem)],
                    acc.at[pl.ds(s * ROWS + full * ZB, rem)])


def _make_deg_kernel():
  @functools.partial(
      pl.kernel,
      out_type=jax.ShapeDtypeStruct((N, DEGW), jnp.float32),
      mesh=_mesh,
      compiler_params=_sc_params,
      name="sc_deg",
      scratch_types=[
          pltpu.VMEM((KD + 1, CHUNK), jnp.int32),
          pltpu.VMEM((CHUNK, DEGW), jnp.float32),
          pltpu.VMEM_SHARED((ACC_N, DEGW), jnp.float32),
          pltpu.SemaphoreType.DMA,
      ],
  )
  def deg_kernel(ei_hbm, ones_hbm, zeros_hbm, out_hbm, dst_v, ones_v, acc,
                 ssem):
    c = lax.axis_index("c")
    s = lax.axis_index("s")

    @pl.when(c == 0)
    def _():
      base = s * KD + jnp.minimum(s, XTRA)
      pltpu.sync_copy(zeros_hbm, acc.at[pl.ds(s * ROWS, ROWS)])
      pltpu.sync_copy(ones_hbm, ones_v)
      pltpu.sync_copy(ei_hbm.at[1, pl.ds(base, KD)], dst_v.at[pl.ds(0, KD)])

      @pl.when(s < XTRA)
      def _():
        pltpu.sync_copy(ei_hbm.at[1, pl.ds(base + KD, 1)],
                        dst_v.at[pl.ds(KD, 1)])

      plsc.subcore_barrier()

      def body(g, carry):
        for b in range(NBUF):
          pltpu.async_copy(ones_v, acc.at[dst_v.at[g * NBUF + b]], ssem,
                           add=True)
        for b in range(NBUF):
          pltpu.make_async_copy(ones_v, acc.at[pl.ds(0, CHUNK)], ssem).wait()
        return carry

      lax.fori_loop(0, KD // NBUF, body, 0)

      @pl.when(s < XTRA)
      def _():
        pltpu.sync_copy(ones_v, acc.at[dst_v.at[KD]], add=True)

      plsc.subcore_barrier()
      _copy_out(acc, out_hbm, s)

  return deg_kernel


def _make_scatter_kernel(d: int):
  @functools.partial(
      pl.kernel,
      out_type=(jax.ShapeDtypeStruct((N, d), jnp.float32),
                jax.ShapeDtypeStruct((N, d), jnp.float32)),
      mesh=_mesh,
      compiler_params=_sc_params,
      name=f"sc_scat{d}",
      scratch_types=[
          pltpu.VMEM((KMAX, CHUNK), jnp.int32),
          pltpu.VMEM((KMAX, CHUNK), jnp.int32),
          pltpu.VMEM((NBUF, CHUNK, d), jnp.float32),
          pltpu.VMEM((ZB, d), jnp.float32),
          pltpu.VMEM_SHARED((ACC_N, d), jnp.float32),
      ] + [pltpu.SemaphoreType.DMA] * (2 * NBUF),
  )
  def scat_kernel(z_hbm, ei_hbm, out0_hbm, out1_hbm,
                  src_v, dst_v, rows_v, zbuf, acc, *sems):
    gsems, ssems = sems[:NBUF], sems[NBUF:]
    c = lax.axis_index("c")
    s = lax.axis_index("s")

    def gather_start(b, j):
      pltpu.async_copy(z_hbm.at[src_v.at[j]], rows_v.at[b], gsems[b])

    def gather_wait(b):
      pltpu.make_async_copy(z_hbm.at[pl.ds(0, CHUNK)], rows_v.at[b],
                            gsems[b]).wait()

    def scatter_start(b, j):
      pltpu.async_copy(rows_v.at[b], acc.at[dst_v.at[j]], ssems[b], add=True)

    def scatter_wait(b):
      pltpu.make_async_copy(rows_v.at[b], acc.at[pl.ds(0, CHUNK)],
                            ssems[b]).wait()

    def run(base, nchunk):
      pltpu.sync_copy(ei_hbm.at[0, pl.ds(base, nchunk)],
                      src_v.at[pl.ds(0, nchunk)])
      pltpu.sync_copy(ei_hbm.at[1, pl.ds(base, nchunk)],
                      dst_v.at[pl.ds(0, nchunk)])
      plsc.subcore_barrier()
      ng = nchunk // NBUF
      for b in range(NBUF):
        gather_start(b, b)

      def body(g, carry):
        for b in range(NBUF):
          gather_wait(b)
          scatter_start(b, g * NBUF + b)
        for b in range(NBUF):
          scatter_wait(b)
          gather_start(b, (g + 1) * NBUF + b)
        return carry

      lax.fori_loop(0, ng - 1, body, 0)
      for b in range(NBUF):
        gather_wait(b)
        scatter_start(b, (ng - 1) * NBUF + b)
      for b in range(NBUF):
        scatter_wait(b)

    @pl.when(c == 0)
    def _():
      base = s * K0 + jnp.minimum(s, XTRA)
      _zero_acc_local(zbuf, acc, s, d)

      @pl.when(s < XTRA)
      def _():
        pltpu.sync_copy(ei_hbm.at[0, pl.ds(base + K0, 1)],
                        src_v.at[pl.ds(K0, 1)])
        pltpu.sync_copy(ei_hbm.at[1, pl.ds(base + K0, 1)],
                        dst_v.at[pl.ds(K0, 1)])

      run(base, K0)

      @pl.when(s < XTRA)
      def _():
        pltpu.sync_copy(z_hbm.at[src_v.at[K0]], rows_v.at[0])
        pltpu.sync_copy(rows_v.at[0], acc.at[dst_v.at[K0]], add=True)

      plsc.subcore_barrier()
      _copy_out(acc, out0_hbm, s)

    @pl.when(c == 1)
    def _():
      _zero_acc_local(zbuf, acc, s, d)
      run(NS * K0 + XTRA + s * K1, K1)
      plsc.subcore_barrier()
      _copy_out(acc, out1_hbm, s)

  return scat_kernel


_deg_kernel = _make_deg_kernel()
_scat32 = _make_scatter_kernel(D_HID)
_scat16 = _make_scatter_kernel(D_OUT)


def _tc1(x_ref, w1_ref, deg_ref, z1_ref, dinv_ref):
  deg = deg_ref[:, :1] + 1.0
  dinv = lax.rsqrt(deg)
  y1 = jnp.dot(x_ref[...], w1_ref[...], preferred_element_type=jnp.float32)
  z1_ref[...] = y1 * dinv
  dinv_ref[...] = dinv


def _tc2(s1a_ref, s1b_ref, z1_ref, dinv_ref, b1_ref, w2_ref, z2_ref):
  dinv = dinv_ref[...]
  h = dinv * (s1a_ref[...] + s1b_ref[...] + z1_ref[...]) + b1_ref[...]
  h = jnp.maximum(h, 0.0)
  y2 = jnp.dot(h, w2_ref[...], preferred_element_type=jnp.float32)
  z2_ref[...] = y2 * dinv


def _tc3(s2a_ref, s2b_ref, z2_ref, dinv_ref, b2_ref, out_ref):
  out_ref[...] = (dinv_ref[...] * (s2a_ref[...] + s2b_ref[...] + z2_ref[...])
                  + b2_ref[...])


def kernel(x, edge_index, W1, b1, W2, b2):
  ei3 = edge_index.reshape(2, TOT_CHUNK, CHUNK)

  ones_c = jnp.ones((CHUNK, DEGW), jnp.float32)
  zeros1 = jnp.zeros((ROWS, DEGW), jnp.float32)

  deg = _deg_kernel(ei3, ones_c, zeros1)

  z1, dinv = pl.pallas_call(
      _tc1,
      out_shape=(jax.ShapeDtypeStruct((N, D_HID), jnp.float32),
                 jax.ShapeDtypeStruct((N, 1), jnp.float32)),
  )(x, W1, deg)

  s1a, s1b = _scat32(z1, ei3)

  z2 = pl.pallas_call(
      _tc2,
      out_shape=jax.ShapeDtypeStruct((N, D_OUT), jnp.float32),
  )(s1a, s1b, z1, dinv, b1, W2)

  s2a, s2b = _scat16(z2, ei3)

  out = pl.pallas_call(
      _tc3,
      out_shape=jax.ShapeDtypeStruct((N, D_OUT), jnp.float32),
  )(s2a, s2b, z2, dinv, b2)

  return out


# speedup vs baseline: 1.0305x; 1.0185x over previous
import functools

import jax
import jax.numpy as jnp
from jax import lax
from jax.experimental import pallas as pl
from jax.experimental.pallas import tpu as pltpu
from jax.experimental.pallas import tpu_sc as plsc

N = 10000
E = 320000
D_IN, D_HID, D_OUT = 128, 32, 16

NC, NS = 2, 16
CHUNK = 128
NBUF = 6
TOT_CHUNK = E // CHUNK
K0 = 72
K1 = 84
XTRA = 4
KMAX = max(K0 + 1, K1)
KD = 156
ROWS = 632
ACC_N = ROWS * NS
TAIL = N - (NS - 1) * ROWS
DEGW = 8
ZB = 64

_mesh = plsc.VectorSubcoreMesh(core_axis_name="c", subcore_axis_name="s")
_sc_params = pltpu.CompilerParams(use_tc_tiling_on_sc=False)


def _copy_out(acc, out_hbm, s):
  @pl.when(s < NS - 1)
  def _():
    pltpu.sync_copy(acc.at[pl.ds(s * ROWS, ROWS)],
                    out_hbm.at[pl.ds(s * ROWS, ROWS)])

  @pl.when(s == NS - 1)
  def _():
    pltpu.sync_copy(acc.at[pl.ds((NS - 1) * ROWS, TAIL)],
                    out_hbm.at[pl.ds((NS - 1) * ROWS, TAIL)])


def _zero_acc_local(zbuf, acc, s, d):
  z16 = jnp.zeros((16,), jnp.float32)

  def zrow(r, carry):
    for cc in range(d // 16):
      zbuf[r, pl.ds(cc * 16, 16)] = z16
    return carry

  lax.fori_loop(0, ZB, zrow, 0)
  full, rem = divmod(ROWS, ZB)
  for i in range(full):
    pltpu.sync_copy(zbuf, acc.at[pl.ds(s * ROWS + i * ZB, ZB)])
  if rem:
    pltpu.sync_copy(zbuf.at[pl.ds(0, rem)],
                    acc.at[pl.ds(s * ROWS + full * ZB, rem)])


def _make_deg_kernel():
  @functools.partial(
      pl.kernel,
      out_type=jax.ShapeDtypeStruct((N, DEGW), jnp.float32),
      mesh=_mesh,
      compiler_params=_sc_params,
      name="sc_deg",
      scratch_types=[
          pltpu.VMEM((KD + 1, CHUNK), jnp.int32),
          pltpu.VMEM((CHUNK, DEGW), jnp.float32),
          pltpu.VMEM_SHARED((ACC_N, DEGW), jnp.float32),
          pltpu.SemaphoreType.DMA,
      ],
  )
  def deg_kernel(ei_hbm, ones_hbm, zeros_hbm, out_hbm, dst_v, ones_v, acc,
                 ssem):
    c = lax.axis_index("c")
    s = lax.axis_index("s")

    @pl.when(c == 0)
    def _():
      base = s * KD + jnp.minimum(s, XTRA)
      pltpu.sync_copy(zeros_hbm, acc.at[pl.ds(s * ROWS, ROWS)])
      pltpu.sync_copy(ones_hbm, ones_v)
      pltpu.sync_copy(ei_hbm.at[1, pl.ds(base, KD)], dst_v.at[pl.ds(0, KD)])

      @pl.when(s < XTRA)
      def _():
        pltpu.sync_copy(ei_hbm.at[1, pl.ds(base + KD, 1)],
                        dst_v.at[pl.ds(KD, 1)])

      plsc.subcore_barrier()

      def body(g, carry):
        for b in range(NBUF):
          pltpu.async_copy(ones_v, acc.at[dst_v.at[g * NBUF + b]], ssem,
                           add=True)
        for b in range(NBUF):
          pltpu.make_async_copy(ones_v, acc.at[pl.ds(0, CHUNK)], ssem).wait()
        return carry

      lax.fori_loop(0, KD // NBUF, body, 0)

      @pl.when(s < XTRA)
      def _():
        pltpu.sync_copy(ones_v, acc.at[dst_v.at[KD]], add=True)

      plsc.subcore_barrier()
      _copy_out(acc, out_hbm, s)

  return deg_kernel


def _make_scatter_kernel(d: int):
  @functools.partial(
      pl.kernel,
      out_type=(jax.ShapeDtypeStruct((N, d), jnp.float32),
                jax.ShapeDtypeStruct((N, d), jnp.float32)),
      mesh=_mesh,
      compiler_params=_sc_params,
      name=f"sc_scat{d}",
      scratch_types=[
          pltpu.VMEM((KMAX, CHUNK), jnp.int32),
          pltpu.VMEM((KMAX, CHUNK), jnp.int32),
          pltpu.VMEM((NBUF, CHUNK, d), jnp.float32),
          pltpu.VMEM((ZB, d), jnp.float32),
          pltpu.VMEM_SHARED((ACC_N, d), jnp.float32),
      ] + [pltpu.SemaphoreType.DMA] * (2 * NBUF),
  )
  def scat_kernel(z_hbm, ei_hbm, out0_hbm, out1_hbm,
                  src_v, dst_v, rows_v, zbuf, acc, *sems):
    gsems, ssems = sems[:NBUF], sems[NBUF:]
    c = lax.axis_index("c")
    s = lax.axis_index("s")

    def gather_start(b, j):
      pltpu.async_copy(z_hbm.at[src_v.at[j]], rows_v.at[b], gsems[b])

    def gather_wait(b):
      pltpu.make_async_copy(z_hbm.at[pl.ds(0, CHUNK)], rows_v.at[b],
                            gsems[b]).wait()

    def scatter_start(b, j):
      pltpu.async_copy(rows_v.at[b], acc.at[dst_v.at[j]], ssems[b], add=True)

    def scatter_wait(b):
      pltpu.make_async_copy(rows_v.at[b], acc.at[pl.ds(0, CHUNK)],
                            ssems[b]).wait()

    def run(base, nchunk):
      pltpu.sync_copy(ei_hbm.at[0, pl.ds(base, nchunk)],
                      src_v.at[pl.ds(0, nchunk)])
      pltpu.sync_copy(ei_hbm.at[1, pl.ds(base, nchunk)],
                      dst_v.at[pl.ds(0, nchunk)])
      plsc.subcore_barrier()
      ng = nchunk // NBUF
      for b in range(NBUF):
        gather_start(b, b)

      def body(g, carry):
        for b in range(NBUF):
          gather_wait(b)
          scatter_start(b, g * NBUF + b)
        for b in range(NBUF):
          scatter_wait(b)
          gather_start(b, (g + 1) * NBUF + b)
        return carry

      lax.fori_loop(0, ng - 1, body, 0)
      for b in range(NBUF):
        gather_wait(b)
        scatter_start(b, (ng - 1) * NBUF + b)
      for b in range(NBUF):
        scatter_wait(b)

    @pl.when(c == 0)
    def _():
      base = s * K0 + jnp.minimum(s, XTRA)
      _zero_acc_local(zbuf, acc, s, d)

      @pl.when(s < XTRA)
      def _():
        pltpu.sync_copy(ei_hbm.at[0, pl.ds(base + K0, 1)],
                        src_v.at[pl.ds(K0, 1)])
        pltpu.sync_copy(ei_hbm.at[1, pl.ds(base + K0, 1)],
                        dst_v.at[pl.ds(K0, 1)])

      run(base, K0)

      @pl.when(s < XTRA)
      def _():
        pltpu.sync_copy(z_hbm.at[src_v.at[K0]], rows_v.at[0])
        pltpu.sync_copy(rows_v.at[0], acc.at[dst_v.at[K0]], add=True)

      plsc.subcore_barrier()
      _copy_out(acc, out0_hbm, s)

    @pl.when(c == 1)
    def _():
      _zero_acc_local(zbuf, acc, s, d)
      run(NS * K0 + XTRA + s * K1, K1)
      plsc.subcore_barrier()
      _copy_out(acc, out1_hbm, s)

  return scat_kernel


_deg_kernel = _make_deg_kernel()
_scat32 = _make_scatter_kernel(D_HID)
_scat16 = _make_scatter_kernel(D_OUT)


def _tc1(x_ref, w1_ref, deg_ref, z1_ref, dinv_ref):
  deg = deg_ref[:, :1] + 1.0
  dinv = lax.rsqrt(deg)
  y1 = jnp.dot(x_ref[...], w1_ref[...], preferred_element_type=jnp.float32)
  z1_ref[...] = y1 * dinv
  dinv_ref[...] = dinv


def _tc2(s1a_ref, s1b_ref, z1_ref, dinv_ref, b1_ref, w2_ref, z2_ref):
  dinv = dinv_ref[...]
  h = dinv * (s1a_ref[...] + s1b_ref[...] + z1_ref[...]) + b1_ref[...]
  h = jnp.maximum(h, 0.0)
  y2 = jnp.dot(h, w2_ref[...], preferred_element_type=jnp.float32)
  z2_ref[...] = y2 * dinv


def _tc3(s2a_ref, s2b_ref, z2_ref, dinv_ref, b2_ref, out_ref):
  out_ref[...] = (dinv_ref[...] * (s2a_ref[...] + s2b_ref[...] + z2_ref[...])
                  + b2_ref[...])


def kernel(x, edge_index, W1, b1, W2, b2):
  ei3 = edge_index.reshape(2, TOT_CHUNK, CHUNK)

  ones_c = jnp.ones((CHUNK, DEGW), jnp.float32)
  zeros1 = jnp.zeros((ROWS, DEGW), jnp.float32)

  deg = _deg_kernel(ei3, ones_c, zeros1)

  z1, dinv = pl.pallas_call(
      _tc1,
      out_shape=(jax.ShapeDtypeStruct((N, D_HID), jnp.float32),
                 jax.ShapeDtypeStruct((N, 1), jnp.float32)),
  )(x, W1, deg)

  s1a, s1b = _scat32(z1, ei3)

  z2 = pl.pallas_call(
      _tc2,
      out_shape=jax.ShapeDtypeStruct((N, D_OUT), jnp.float32),
  )(s1a, s1b, z1, dinv, b1, W2)

  s2a, s2b = _scat16(z2, ei3)

  out = pl.pallas_call(
      _tc3,
      out_shape=jax.ShapeDtypeStruct((N, D_OUT), jnp.float32),
  )(s2a, s2b, z2, dinv, b2)

  return out
